# Initial kernel scaffold; baseline (speedup 1.0000x reference)
import jax
import jax.numpy as jnp
from jax.experimental import pallas as pl


def kernel(H, Z, block_id, batch_id, edges, edge_attr, W_in, b_in, W_e1, b_e1, W_e2, b_e2, W_x1, b_x1, W_x2, W_h1, b_h1, W_h2, b_h2, W_out, b_out):
    raise NotImplementedError("write your pallas kernel here")


# speedup vs baseline: 1.3223x; 1.3223x over previous
import functools
import jax
import jax.numpy as jnp
from jax.experimental import pallas as pl
from jax.experimental.pallas import tpu as pltpu

N = 50000
BS = 4
NB = N // BS
E = 200000
D = 128
DE = 16
L = 2
K = 3
EK = E * K

TE = 2000
TN = 2000


def _silu(x):
    return x * jax.nn.sigmoid(x)


def _edge_body(hr, hc, ea, xd, A, B, C, W2, Wx1, P, m_out, dx_out):
    b1 = P[0:1, :]
    b2 = P[1:2, :]
    bx = P[2:3, :]
    wr = P[3:4, :]
    wx2 = P[4:5, :]
    xdv = xd[...]
    radial = jnp.sum(xdv * xdv, axis=1, keepdims=True)
    z = jnp.dot(hr[...], A[...], preferred_element_type=jnp.float32)
    z = z + jnp.dot(hc[...], B[...], preferred_element_type=jnp.float32)
    z = z + jnp.dot(ea[...], C[...], preferred_element_type=jnp.float32)
    z = z + radial * wr
    m1 = _silu(z + b1)
    m = _silu(jnp.dot(m1, W2[...], preferred_element_type=jnp.float32) + b2)
    m_out[...] = m
    tp = _silu(jnp.dot(m, Wx1[...], preferred_element_type=jnp.float32) + bx)
    t = jnp.sum(tp * wx2, axis=1, keepdims=True)
    dx_out[...] = xdv * t


def _edge_mlp(hr, hc, ea, xd, A, B, C, W2, Wx1, P):
    grid = (EK // TE,)
    ew = lambda i: (i, 0)
    w0 = lambda i: (0, 0)
    return pl.pallas_call(
        _edge_body,
        grid=grid,
        in_specs=[
            pl.BlockSpec((TE, D), ew),
            pl.BlockSpec((TE, D), ew),
            pl.BlockSpec((TE, DE), ew),
            pl.BlockSpec((TE, 8), ew),
            pl.BlockSpec((D, D), w0),
            pl.BlockSpec((D, D), w0),
            pl.BlockSpec((DE, D), w0),
            pl.BlockSpec((D, D), w0),
            pl.BlockSpec((D, D), w0),
            pl.BlockSpec((8, D), w0),
        ],
        out_specs=[
            pl.BlockSpec((TE, D), ew),
            pl.BlockSpec((TE, 8), ew),
        ],
        out_shape=[
            jax.ShapeDtypeStruct((EK, D), jnp.float32),
            jax.ShapeDtypeStruct((EK, 8), jnp.float32),
        ],
    )(hr, hc, ea, xd, A, B, C, W2, Wx1, P)


def _node_body(h, agg, xp, xaggp, cntp, Wa, Wb, W2, P, h_out, x_out):
    b1 = P[0:1, :]
    b2 = P[1:2, :]
    hv = h[...]
    z = jnp.dot(hv, Wa[...], preferred_element_type=jnp.float32)
    z = z + jnp.dot(agg[...], Wb[...], preferred_element_type=jnp.float32)
    u = _silu(z + b1)
    h_out[...] = hv + jnp.dot(u, W2[...], preferred_element_type=jnp.float32) + b2
    x_out[...] = xp[...] + xaggp[...] / cntp[...]


def _node_mlp(h, agg, xp, xaggp, cntp, Wa, Wb, W2, P):
    grid = (N // TN,)
    nw = lambda i: (i, 0)
    w0 = lambda i: (0, 0)
    return pl.pallas_call(
        _node_body,
        grid=grid,
        in_specs=[
            pl.BlockSpec((TN, D), nw),
            pl.BlockSpec((TN, D), nw),
            pl.BlockSpec((TN, 8), nw),
            pl.BlockSpec((TN, 8), nw),
            pl.BlockSpec((TN, 8), nw),
            pl.BlockSpec((D, D), w0),
            pl.BlockSpec((D, D), w0),
            pl.BlockSpec((D, D), w0),
            pl.BlockSpec((8, D), w0),
        ],
        out_specs=[
            pl.BlockSpec((TN, D), nw),
            pl.BlockSpec((TN, 8), nw),
        ],
        out_shape=[
            jax.ShapeDtypeStruct((N, D), jnp.float32),
            jax.ShapeDtypeStruct((N, 8), jnp.float32),
        ],
    )(h, agg, xp, xaggp, cntp, Wa, Wb, W2, P)


def _proj_body(h, W, P, out):
    out[...] = jnp.dot(h[...], W[...], preferred_element_type=jnp.float32) + P[0:1, :]


def _proj(h, W, P):
    grid = (N // TN,)
    nw = lambda i: (i, 0)
    w0 = lambda i: (0, 0)
    return pl.pallas_call(
        _proj_body,
        grid=grid,
        in_specs=[
            pl.BlockSpec((TN, D), nw),
            pl.BlockSpec((D, D), w0),
            pl.BlockSpec((8, D), w0),
        ],
        out_specs=pl.BlockSpec((TN, D), nw),
        out_shape=jax.ShapeDtypeStruct((N, D), jnp.float32),
    )(h, W, P)


def _pad8(b):
    return jnp.concatenate([b.reshape(1, -1), jnp.zeros((7, b.shape[-1]), jnp.float32)], axis=0)


def kernel(H, Z, block_id, batch_id, edges, edge_attr, W_in, b_in, W_e1, b_e1, W_e2, b_e2,
           W_x1, b_x1, W_x2, W_h1, b_h1, W_h2, b_h2, W_out, b_out):
    offs = jnp.arange(BS, dtype=edges.dtype)
    src_atoms = edges[0][:, None] * BS + offs[None, :]
    dst_atoms = edges[1][:, None] * BS + offs[None, :]
    ps = Z[src_atoms]
    pd = Z[dst_atoms]
    d2 = jnp.sum((ps[:, :, None, :] - pd[:, None, :, :]) ** 2, axis=-1).reshape(E, BS * BS)
    _, idx = jax.lax.top_k(-d2, K)
    a = idx // BS
    b = idx % BS
    row = (edges[0][:, None] * BS + a).reshape(-1)
    col = (edges[1][:, None] * BS + b).reshape(-1)
    ea = jnp.repeat(edge_attr, K, axis=0)

    h = _proj(H, W_in, _pad8(b_in))
    x8 = jnp.concatenate([Z, jnp.zeros((N, 5), jnp.float32)], axis=1)
    ones = jnp.ones((EK,), jnp.float32)
    cnt = jnp.maximum(jax.ops.segment_sum(ones, row, num_segments=N), 1.0)
    cnt8 = jnp.broadcast_to(cnt[:, None], (N, 8))

    for l in range(L):
        A = W_e1[l, :D]
        B = W_e1[l, D:2 * D]
        wr = W_e1[l, 2 * D:2 * D + 1]
        C = W_e1[l, 2 * D + 1:]
        P_e = jnp.concatenate([
            b_e1[l].reshape(1, D), b_e2[l].reshape(1, D), b_x1[l].reshape(1, D),
            wr.reshape(1, D), W_x2[l].reshape(1, D), jnp.zeros((3, D), jnp.float32)
        ], axis=0)
        hr = h[row]
        hc = h[col]
        xd = x8[row] - x8[col]
        m, dx = _edge_mlp(hr, hc, ea, xd, A, B, C, W_e2[l], W_x1[l], P_e)
        agg = jax.ops.segment_sum(m, row, num_segments=N)
        xagg = jax.ops.segment_sum(dx, row, num_segments=N)
        P_h = jnp.concatenate([
            b_h1[l].reshape(1, D), b_h2[l].reshape(1, D), jnp.zeros((6, D), jnp.float32)
        ], axis=0)
        h, x8 = _node_mlp(h, agg, x8, xagg, cnt8, W_h1[l, :D], W_h1[l, D:], W_h2[l], P_h)

    h = _proj(h, W_out, _pad8(b_out))
    return (h, x8[:, :3])


# speedup vs baseline: 1.7769x; 1.3438x over previous
import functools
import jax
import jax.numpy as jnp
from jax.experimental import pallas as pl
from jax.experimental.pallas import tpu as pltpu
from jax.experimental.pallas import tpu_sc as plsc

N = 50000
BS = 4
NB = N // BS
E = 200000
D = 128
DE = 16
L = 2
K = 3
EK = E * K

TE = 2000
TN = 2000


def _silu(x):
    return x * jax.nn.sigmoid(x)


_NC = 2
_NW = 32
BP = 614400
BPW = BP // _NW


def _sc_gather(table, idx, width, ch):
    nchunk = BPW // ch
    mesh = plsc.VectorSubcoreMesh(core_axis_name="c", subcore_axis_name="s")

    @functools.partial(
        pl.kernel, mesh=mesh,
        out_type=jax.ShapeDtypeStruct((BP, width), jnp.float32),
        scratch_types=[
            pltpu.VMEM((ch,), jnp.int32),
            pltpu.VMEM((ch, width), jnp.float32),
            pltpu.SemaphoreType.DMA,
        ],
    )
    def gk(t_h, i_h, out_h, idx_v, rows_v, sem):
        wid = jax.lax.axis_index("s") * _NC + jax.lax.axis_index("c")
        base = wid * BPW

        def body(c, carry):
            off = pl.multiple_of(base + c * ch, ch)
            pltpu.sync_copy(i_h.at[pl.ds(off, ch)], idx_v)
            pltpu.async_copy(t_h.at[idx_v], rows_v, sem).wait()
            pltpu.sync_copy(rows_v, out_h.at[pl.ds(off, ch)])
            return carry

        jax.lax.fori_loop(0, nchunk, body, 0)

    return gk(table, idx)


def _pad_idx(i):
    return jnp.concatenate([i, jnp.zeros((BP - EK,), jnp.int32)])


def _edge_body(hra, hcb, ea, xd, C, W2, Wx1, P, m_out, dx_out):
    b1 = P[0:1, :]
    b2 = P[1:2, :]
    bx = P[2:3, :]
    wr = P[3:4, :]
    wx2 = P[4:5, :]
    xdv = xd[...]
    radial = jnp.sum(xdv * xdv, axis=1, keepdims=True)
    z = hra[...] + hcb[...]
    z = z + jnp.dot(ea[...], C[...], preferred_element_type=jnp.float32)
    z = z + radial * wr
    m1 = _silu(z + b1)
    m = _silu(jnp.dot(m1, W2[...], preferred_element_type=jnp.float32) + b2)
    m_out[...] = m
    tp = _silu(jnp.dot(m, Wx1[...], preferred_element_type=jnp.float32) + bx)
    t = jnp.sum(tp * wx2, axis=1, keepdims=True)
    dx_out[...] = xdv * t


def _edge_mlp(hra, hcb, ea, xd, C, W2, Wx1, P):
    grid = (EK // TE,)
    ew = lambda i: (i, 0)
    w0 = lambda i: (0, 0)
    return pl.pallas_call(
        _edge_body,
        grid=grid,
        in_specs=[
            pl.BlockSpec((TE, D), ew),
            pl.BlockSpec((TE, D), ew),
            pl.BlockSpec((TE, DE), ew),
            pl.BlockSpec((TE, 16), ew),
            pl.BlockSpec((DE, D), w0),
            pl.BlockSpec((D, D), w0),
            pl.BlockSpec((D, D), w0),
            pl.BlockSpec((8, D), w0),
        ],
        out_specs=[
            pl.BlockSpec((TE, D), ew),
            pl.BlockSpec((TE, 16), ew),
        ],
        out_shape=[
            jax.ShapeDtypeStruct((EK, D), jnp.float32),
            jax.ShapeDtypeStruct((EK, 16), jnp.float32),
        ],
    )(hra, hcb, ea, xd, C, W2, Wx1, P)


def _node_body(h, agg, xp, xaggp, cntp, Wa, Wb, W2, P, h_out, x_out):
    b1 = P[0:1, :]
    b2 = P[1:2, :]
    hv = h[...]
    z = jnp.dot(hv, Wa[...], preferred_element_type=jnp.float32)
    z = z + jnp.dot(agg[...], Wb[...], preferred_element_type=jnp.float32)
    u = _silu(z + b1)
    h_out[...] = hv + jnp.dot(u, W2[...], preferred_element_type=jnp.float32) + b2
    x_out[...] = xp[...] + xaggp[...] / cntp[...]


def _node_mlp(h, agg, xp, xaggp, cntp, Wa, Wb, W2, P):
    grid = (N // TN,)
    nw = lambda i: (i, 0)
    w0 = lambda i: (0, 0)
    return pl.pallas_call(
        _node_body,
        grid=grid,
        in_specs=[
            pl.BlockSpec((TN, D), nw),
            pl.BlockSpec((TN, D), nw),
            pl.BlockSpec((TN, 16), nw),
            pl.BlockSpec((TN, 16), nw),
            pl.BlockSpec((TN, 16), nw),
            pl.BlockSpec((D, D), w0),
            pl.BlockSpec((D, D), w0),
            pl.BlockSpec((D, D), w0),
            pl.BlockSpec((8, D), w0),
        ],
        out_specs=[
            pl.BlockSpec((TN, D), nw),
            pl.BlockSpec((TN, 16), nw),
        ],
        out_shape=[
            jax.ShapeDtypeStruct((N, D), jnp.float32),
            jax.ShapeDtypeStruct((N, 16), jnp.float32),
        ],
    )(h, agg, xp, xaggp, cntp, Wa, Wb, W2, P)


def _proj_body(h, W, P, out):
    out[...] = jnp.dot(h[...], W[...], preferred_element_type=jnp.float32) + P[0:1, :]


def _proj(h, W, P):
    R, Win = h.shape
    grid = (R // TN,)
    nw = lambda i: (i, 0)
    w0 = lambda i: (0, 0)
    return pl.pallas_call(
        _proj_body,
        grid=grid,
        in_specs=[
            pl.BlockSpec((TN, Win), nw),
            pl.BlockSpec((Win, D), w0),
            pl.BlockSpec((8, D), w0),
        ],
        out_specs=pl.BlockSpec((TN, D), nw),
        out_shape=jax.ShapeDtypeStruct((R, D), jnp.float32),
    )(h, W, P)


def _pad8(b):
    return jnp.concatenate([b.reshape(1, -1), jnp.zeros((7, b.shape[-1]), jnp.float32)], axis=0)


def kernel(H, Z, block_id, batch_id, edges, edge_attr, W_in, b_in, W_e1, b_e1, W_e2, b_e2,
           W_x1, b_x1, W_x2, W_h1, b_h1, W_h2, b_h2, W_out, b_out):
    offs = jnp.arange(BS, dtype=edges.dtype)
    src_atoms = edges[0][:, None] * BS + offs[None, :]
    dst_atoms = edges[1][:, None] * BS + offs[None, :]
    ps = Z[src_atoms]
    pd = Z[dst_atoms]
    d2 = jnp.sum((ps[:, :, None, :] - pd[:, None, :, :]) ** 2, axis=-1).reshape(E, BS * BS)
    _, idx = jax.lax.top_k(-d2, K)
    a = idx // BS
    b = idx % BS
    row = (edges[0][:, None] * BS + a).reshape(-1)
    col = (edges[1][:, None] * BS + b).reshape(-1)
    eid = jnp.repeat(jnp.arange(E, dtype=jnp.int32), K)

    rowp = _pad_idx(row)
    colp = _pad_idx(col)
    ea = jnp.repeat(edge_attr, K, axis=0)

    h = _proj(H, W_in, _pad8(b_in))
    x16 = jnp.concatenate([Z, jnp.zeros((N, 13), jnp.float32)], axis=1)
    ones = jnp.ones((EK,), jnp.float32)
    cnt = jnp.maximum(jax.ops.segment_sum(ones, row, num_segments=N), 1.0)
    cnt16 = jnp.broadcast_to(cnt[:, None], (N, 16))

    zero8 = jnp.zeros((8, D), jnp.float32)

    for l in range(L):
        A = W_e1[l, :D]
        B = W_e1[l, D:2 * D]
        wr = W_e1[l, 2 * D:2 * D + 1]
        C = W_e1[l, 2 * D + 1:]
        P_e = jnp.concatenate([
            b_e1[l].reshape(1, D), b_e2[l].reshape(1, D), b_x1[l].reshape(1, D),
            wr.reshape(1, D), W_x2[l].reshape(1, D), jnp.zeros((3, D), jnp.float32)
        ], axis=0)
        hA = _proj(h, A, zero8)
        hB = _proj(h, B, zero8)
        hra = _sc_gather(hA, rowp, D, 320)
        hcb = _sc_gather(hB, colp, D, 320)
        xd = x16[row] - x16[col]
        m, dx = _edge_mlp(hra, hcb, ea, xd, C, W_e2[l], W_x1[l], P_e)
        agg = jax.ops.segment_sum(m, row, num_segments=N)
        xagg = jax.ops.segment_sum(dx[:EK], row, num_segments=N)
        P_h = jnp.concatenate([
            b_h1[l].reshape(1, D), b_h2[l].reshape(1, D), jnp.zeros((6, D), jnp.float32)
        ], axis=0)
        h, x16 = _node_mlp(h, agg, x16, xagg, cnt16, W_h1[l, :D], W_h1[l, D:], W_h2[l], P_h)

    h = _proj(h, W_out, _pad8(b_out))
    return (h, x16[:, :3])


# speedup vs baseline: 1.9290x; 1.0856x over previous
import functools
import jax
import jax.numpy as jnp
from jax.experimental import pallas as pl
from jax.experimental.pallas import tpu as pltpu
from jax.experimental.pallas import tpu_sc as plsc

N = 50000
BS = 4
NB = N // BS
E = 200000
D = 128
DE = 16
L = 2
K = 3
EK = E * K

TE = 2000
TN = 2000


def _silu(x):
    return x * jax.nn.sigmoid(x)


_NC = 2
_NW = 32
BP = 614400
BPW = BP // _NW


def _sc_gather(table, idx, width, ch):
    nchunk = BPW // ch
    mesh = plsc.VectorSubcoreMesh(core_axis_name="c", subcore_axis_name="s")

    @functools.partial(
        pl.kernel, mesh=mesh,
        out_type=jax.ShapeDtypeStruct((BP, width), jnp.float32),
        scratch_types=[
            pltpu.VMEM((ch,), jnp.int32),
            pltpu.VMEM((ch, width), jnp.float32),
            pltpu.SemaphoreType.DMA,
        ],
    )
    def gk(t_h, i_h, out_h, idx_v, rows_v, sem):
        wid = jax.lax.axis_index("s") * _NC + jax.lax.axis_index("c")
        base = wid * BPW

        def body(c, carry):
            off = pl.multiple_of(base + c * ch, ch)
            pltpu.sync_copy(i_h.at[pl.ds(off, ch)], idx_v)
            pltpu.async_copy(t_h.at[idx_v], rows_v, sem).wait()
            pltpu.sync_copy(rows_v, out_h.at[pl.ds(off, ch)])
            return carry

        jax.lax.fori_loop(0, nchunk, body, 0)

    return gk(table, idx)


def _pad_idx(i):
    return jnp.concatenate([i, jnp.zeros((BP - EK,), jnp.int32)])


TEB = 1000
NEB = E // TEB


def _edge_body(hra0, hra1, hra2, hcb0, hcb1, hcb2, ea, xd0, xd1, xd2,
               oh0, oh1, oh2, C, W2, Wx1, P, m4_out, dx4_out):
    b1 = P[0:1, :]
    b2 = P[1:2, :]
    bx = P[2:3, :]
    wr = P[3:4, :]
    wx2 = P[4:5, :]
    eaC = jnp.dot(ea[...], C[...], preferred_element_type=jnp.float32)
    lane8 = (jax.lax.broadcasted_iota(jnp.int32, (1, 16), 1) == 8).astype(jnp.float32)
    hras = (hra0, hra1, hra2)
    hcbs = (hcb0, hcb1, hcb2)
    xds = (xd0, xd1, xd2)
    ohs = (oh0, oh1, oh2)
    ms = []
    dxs = []
    for k in range(K):
        xdv = xds[k][...]
        radial = jnp.sum(xdv * xdv, axis=1, keepdims=True)
        z = hras[k][...] + hcbs[k][...] + eaC + radial * wr
        m1 = _silu(z + b1)
        m = _silu(jnp.dot(m1, W2[...], preferred_element_type=jnp.float32) + b2)
        tp = _silu(jnp.dot(m, Wx1[...], preferred_element_type=jnp.float32) + bx)
        t = jnp.sum(tp * wx2, axis=1, keepdims=True)
        ms.append(m)
        dxs.append(xdv * t + lane8)
    for j in range(BS):
        mj = ms[0] * ohs[0][:, j:j + 1]
        dj = dxs[0] * ohs[0][:, j:j + 1]
        for k in range(1, K):
            mj = mj + ms[k] * ohs[k][:, j:j + 1]
            dj = dj + dxs[k] * ohs[k][:, j:j + 1]
        m4_out[:, j * D:(j + 1) * D] = mj
        dx4_out[:, j * 16:(j + 1) * 16] = dj


def _edge_mlp(hra, hcb, edge_attr, xd, oh, C, W2, Wx1, P):
    grid = (NEB,)
    ew = lambda i: (i, 0)
    w0 = lambda i: (0, 0)
    ks = lambda k: (lambda i: (k * NEB + i, 0))
    in_specs = (
        [pl.BlockSpec((TEB, D), ks(k)) for k in range(K)]
        + [pl.BlockSpec((TEB, D), ks(k)) for k in range(K)]
        + [pl.BlockSpec((TEB, DE), ew)]
        + [pl.BlockSpec((TEB, 16), ks(k)) for k in range(K)]
        + [pl.BlockSpec((TEB, 8), ks(k)) for k in range(K)]
        + [
            pl.BlockSpec((DE, D), w0),
            pl.BlockSpec((D, D), w0),
            pl.BlockSpec((D, D), w0),
            pl.BlockSpec((8, D), w0),
        ]
    )
    return pl.pallas_call(
        _edge_body,
        grid=grid,
        in_specs=in_specs,
        out_specs=[
            pl.BlockSpec((TEB, BS * D), ew),
            pl.BlockSpec((TEB, BS * 16), ew),
        ],
        out_shape=[
            jax.ShapeDtypeStruct((E, BS * D), jnp.float32),
            jax.ShapeDtypeStruct((E, BS * 16), jnp.float32),
        ],
    )(hra, hra, hra, hcb, hcb, hcb, edge_attr, xd, xd, xd, oh, oh, oh,
      C, W2, Wx1, P)


def _node_body(h, agg, xp, xaggp, cntp, Wa, Wb, W2, P, h_out, x_out):
    b1 = P[0:1, :]
    b2 = P[1:2, :]
    hv = h[...]
    z = jnp.dot(hv, Wa[...], preferred_element_type=jnp.float32)
    z = z + jnp.dot(agg[...], Wb[...], preferred_element_type=jnp.float32)
    u = _silu(z + b1)
    h_out[...] = hv + jnp.dot(u, W2[...], preferred_element_type=jnp.float32) + b2
    not8 = (jax.lax.broadcasted_iota(jnp.int32, (1, 16), 1) != 8).astype(jnp.float32)
    x_out[...] = xp[...] + (xaggp[...] / cntp[...]) * not8


def _node_mlp(h, agg, xp, xaggp, cntp, Wa, Wb, W2, P):
    grid = (N // TN,)
    nw = lambda i: (i, 0)
    w0 = lambda i: (0, 0)
    return pl.pallas_call(
        _node_body,
        grid=grid,
        in_specs=[
            pl.BlockSpec((TN, D), nw),
            pl.BlockSpec((TN, D), nw),
            pl.BlockSpec((TN, 16), nw),
            pl.BlockSpec((TN, 16), nw),
            pl.BlockSpec((TN, 16), nw),
            pl.BlockSpec((D, D), w0),
            pl.BlockSpec((D, D), w0),
            pl.BlockSpec((D, D), w0),
            pl.BlockSpec((8, D), w0),
        ],
        out_specs=[
            pl.BlockSpec((TN, D), nw),
            pl.BlockSpec((TN, 16), nw),
        ],
        out_shape=[
            jax.ShapeDtypeStruct((N, D), jnp.float32),
            jax.ShapeDtypeStruct((N, 16), jnp.float32),
        ],
    )(h, agg, xp, xaggp, cntp, Wa, Wb, W2, P)


def _proj_body(h, W, P, out):
    out[...] = jnp.dot(h[...], W[...], preferred_element_type=jnp.float32) + P[0:1, :]


def _proj(h, W, P):
    R, Win = h.shape
    grid = (R // TN,)
    nw = lambda i: (i, 0)
    w0 = lambda i: (0, 0)
    return pl.pallas_call(
        _proj_body,
        grid=grid,
        in_specs=[
            pl.BlockSpec((TN, Win), nw),
            pl.BlockSpec((Win, D), w0),
            pl.BlockSpec((8, D), w0),
        ],
        out_specs=pl.BlockSpec((TN, D), nw),
        out_shape=jax.ShapeDtypeStruct((R, D), jnp.float32),
    )(h, W, P)


def _pad8(b):
    return jnp.concatenate([b.reshape(1, -1), jnp.zeros((7, b.shape[-1]), jnp.float32)], axis=0)


def kernel(H, Z, block_id, batch_id, edges, edge_attr, W_in, b_in, W_e1, b_e1, W_e2, b_e2,
           W_x1, b_x1, W_x2, W_h1, b_h1, W_h2, b_h2, W_out, b_out):
    offs = jnp.arange(BS, dtype=edges.dtype)
    src_atoms = edges[0][:, None] * BS + offs[None, :]
    dst_atoms = edges[1][:, None] * BS + offs[None, :]
    ps = Z[src_atoms]
    pd = Z[dst_atoms]
    d2 = jnp.sum((ps[:, :, None, :] - pd[:, None, :, :]) ** 2, axis=-1).reshape(E, BS * BS)
    _, idx = jax.lax.top_k(-d2, K)
    a = (idx // BS).T.reshape(-1)
    b = (idx % BS).T.reshape(-1)
    ebk = jnp.tile(edges, (1, K))
    row = ebk[0] * BS + a
    col = ebk[1] * BS + b
    oh = jax.nn.one_hot(a, 4, dtype=jnp.float32)
    oh = jnp.concatenate([oh, jnp.zeros((K * E, 4), jnp.float32)], axis=1)

    rowp = _pad_idx(row)
    colp = _pad_idx(col)

    h = _proj(H, W_in, _pad8(b_in))
    x16 = jnp.concatenate([Z, jnp.zeros((N, 13), jnp.float32)], axis=1)

    zero8 = jnp.zeros((8, D), jnp.float32)

    for l in range(L):
        A = W_e1[l, :D]
        B = W_e1[l, D:2 * D]
        wr = W_e1[l, 2 * D:2 * D + 1]
        C = W_e1[l, 2 * D + 1:]
        P_e = jnp.concatenate([
            b_e1[l].reshape(1, D), b_e2[l].reshape(1, D), b_x1[l].reshape(1, D),
            wr.reshape(1, D), W_x2[l].reshape(1, D), jnp.zeros((3, D), jnp.float32)
        ], axis=0)
        hA = _proj(h, A, zero8)
        hB = _proj(h, B, zero8)
        hra = _sc_gather(hA, rowp, D, 320)
        hcb = _sc_gather(hB, colp, D, 320)
        xd = x16[row] - x16[col]
        m4, dx4 = _edge_mlp(hra, hcb, edge_attr, xd, oh, C, W_e2[l], W_x1[l], P_e)
        agg = jax.ops.segment_sum(m4, edges[0], num_segments=NB).reshape(N, D)
        xagg = jax.ops.segment_sum(dx4, edges[0], num_segments=NB).reshape(N, 16)
        cnt16 = jnp.broadcast_to(jnp.maximum(xagg[:, 8:9], 1.0), (N, 16))
        P_h = jnp.concatenate([
            b_h1[l].reshape(1, D), b_h2[l].reshape(1, D), jnp.zeros((6, D), jnp.float32)
        ], axis=0)
        h, x16 = _node_mlp(h, agg, x16, xagg, cnt16, W_h1[l, :D], W_h1[l, D:], W_h2[l], P_h)

    h = _proj(h, W_out, _pad8(b_out))
    return (h, x16[:, :3])
